# baseline (speedup 1.0000x reference)
import functools
import math

import jax
import jax.numpy as jnp
from jax import lax
from jax.experimental import pallas as pl
from jax.experimental.pallas import tpu as pltpu
from jax.experimental.pallas import tpu_sc as plsc

SC = 128
H = 8
QK = 16
HD = 32
F32 = jnp.float32


def _ln(x, g, b, eps=1e-6):
    dk = x.shape[-1]
    w = jnp.full((dk, 1), 1.0 / dk, F32)
    m = jax.lax.dot(x, w, preferred_element_type=F32)
    m2 = jax.lax.dot(x * x, w, preferred_element_type=F32)
    v = m2 - m * m
    return (x - m) * jax.lax.rsqrt(v + eps) * g + b


def _silu(x):
    return x * jax.nn.sigmoid(x)


def _dot(a, b):
    return jax.lax.dot(a, b, preferred_element_type=F32)


def _node_prep_body(x_ref, lng, lnb, qw1, qb1, qg, qbn, qw2, qb2,
                    kw1, kb1, kg, kbn, kw2, kb2, ws, wd, nsw, nsb,
                    ts_ref, td_ref, ns_ref):
    x = x_ref[...]
    node = _ln(x, lng[...], lnb[...])
    scal = node[:, :SC]

    def qkproj(w1, b1, g, bn, w2, b2):
        h = _silu(_ln(_dot(scal, w1[...]) + b1[...], g[...], bn[...]))
        return _dot(h, w2[...]) + b2[...]

    qn = qkproj(qw1, qb1, qg, qbn, qw2, qb2)
    kn = qkproj(kw1, kb1, kg, kbn, kw2, kb2)
    ts_ref[:, :256] = _dot(node, ws[...])
    ts_ref[:, 256:] = kn
    td_ref[:, :256] = _dot(node, wd[...])
    td_ref[:, 256:] = qn
    ns_ref[...] = _dot(x, nsw[...]) + nsb[...]


def _node_prep(node_fea, p, bn=1000):
    n, dn = node_fea.shape
    grid = n // bn

    def row_spec(d):
        return pl.BlockSpec((bn, d), lambda i: (i, 0))

    def w_spec(a):
        return pl.BlockSpec(a.shape, lambda i: tuple(0 for _ in a.shape))

    r2 = lambda a: a.reshape(1, -1)
    weights = [r2(p["ln_node_g"]), r2(p["ln_node_b"]),
               p["q_w1"], r2(p["q_b1"]), r2(p["q_g"]), r2(p["q_bn"]),
               p["q_w2"], r2(p["q_b2"]),
               p["k_w1"], r2(p["k_b1"]), r2(p["k_g"]), r2(p["k_bn"]),
               p["k_w2"], r2(p["k_b2"]),
               p["pre_w"][:dn], p["pre_w"][dn:2 * dn],
               p["ns_w"], r2(p["ns_b"])]
    return pl.pallas_call(
        _node_prep_body,
        grid=(grid,),
        in_specs=[row_spec(dn)] + [w_spec(a) for a in weights],
        out_specs=[row_spec(384), row_spec(384), row_spec(dn)],
        out_shape=[jax.ShapeDtypeStruct((n, 384), F32),
                   jax.ShapeDtypeStruct((n, 384), F32),
                   jax.ShapeDtypeStruct((n, dn), F32)],
    )(node_fea, *weights)


def _edge_body(ef_ref, el_ref, dm_ref, ev_ref, gs_ref, gd_ref,
               lneg, lneb, aw1, ab1, ag1, abg1, aw2, ab2, ag2, abg2, aw3, ab3,
               we, preb, c1r1, c1r2, c1w, c1d, c1g, c1b,
               c2r1, c2r2, c2w, c2d, onw, onb, linew, lineb, esw, esb,
               sel, expm, emsg_ref, eout_ref):
    ef = ef_ref[...]
    el = el_ref[...]
    edge = _ln(ef, lneg[...], lneb[...])
    h = _silu(_ln(_dot(el, aw1[...]) + ab1[...], ag1[...], abg1[...]))
    h = _silu(_ln(_dot(h, aw2[...]) + ab2[...], ag2[...], abg2[...]))
    bias = _dot(h, aw3[...]) + ab3[...]
    gs = gs_ref[...]
    gd = gd_ref[...]
    qk = _dot(gs[:, 256:] * gd[:, 256:], sel[...]) * (1.0 / math.sqrt(QK))
    alpha = qk + bias
    msg = gs[:, :256] + gd[:, :256] + _dot(edge, we[...]) + preb[...]
    dm = dm_ref[...]
    r1 = _dot(_silu(_dot(el, c1r1[...])), c1r2[...])
    v = _dot(msg * r1, c1w[...]) + _dot(dm, c1d[...])
    v = _silu(_ln(v, c1g[...], c1b[...]))
    r2 = _dot(_silu(_dot(el, c2r1[...])), c2r2[...])
    value = _dot(v * r2, c2w[...]) + _dot(dm, c2d[...])
    av = _dot(alpha, expm[...])
    ev = ev_ref[...]
    ons = (ev[:, 0:1] * ev[:, 0:1] + ev[:, 1:2] * ev[:, 1:2]
           + ev[:, 2:3] * ev[:, 2:3]) < 1e-20
    emsg = value * av
    emsg_ref[...] = emsg.T
    eout_ref[...] = (_dot(emsg, linew[...]) + lineb[...]
                     + _dot(ef, esw[...]) + esb[...])

    @pl.when(jnp.any(ons))
    def _():
        onsite_val = _dot(msg, onw[...]) + onb[...]
        emsg2 = jnp.where(ons, onsite_val, value) * av
        emsg_ref[...] = emsg2.T
        eout_ref[...] = (_dot(emsg2, linew[...]) + lineb[...]
                         + _dot(ef, esw[...]) + esb[...])


def _edge_stage(edge_fea, elen, dm, edge_vec, gs, gd, p, be=1280):
    e, de = edge_fea.shape
    grid = e // be
    sel = (jnp.arange(SC)[:, None] // QK == jnp.arange(H)[None, :]).astype(F32)
    expm = (jnp.arange(H)[:, None] == jnp.arange(H * HD)[None, :] // HD).astype(F32)

    def row_spec(d):
        return pl.BlockSpec((be, d), lambda i: (i, 0))

    def w_spec(a):
        return pl.BlockSpec(a.shape, lambda i: tuple(0 for _ in a.shape))

    r2 = lambda a: a.reshape(1, -1)
    weights = [r2(p["ln_edge_g"]), r2(p["ln_edge_b"]),
               p["a_w1"], r2(p["a_b1"]), r2(p["a_g1"]), r2(p["a_bg1"]),
               p["a_w2"], r2(p["a_b2"]), r2(p["a_g2"]), r2(p["a_bg2"]),
               p["a_w3"], r2(p["a_b3"]),
               p["pre_w"][512:], r2(p["pre_b"]),
               p["c1_r1"], p["c1_r2"], p["c1_w"], p["c1_d"],
               r2(p["c1_g"]), r2(p["c1_b"]),
               p["c2_r1"], p["c2_r2"], p["c2_w"], p["c2_d"],
               p["on_w"], r2(p["on_b"]),
               p["line_w"], r2(p["line_b"]), p["es_w"], r2(p["es_b"]),
               sel, expm]
    return pl.pallas_call(
        _edge_body,
        grid=(grid,),
        in_specs=[row_spec(de), row_spec(64), row_spec(9), row_spec(3),
                  row_spec(384), row_spec(384)] + [w_spec(a) for a in weights],
        out_specs=[pl.BlockSpec((256, be), lambda i: (0, i)), row_spec(de)],
        out_shape=[jax.ShapeDtypeStruct((256, e), F32),
                   jax.ShapeDtypeStruct((e, de), F32)],
    )(edge_fea, elen, dm, edge_vec, gs, gd, *weights)


def _sc_gather(table_s, table_d, edge_src, edge_dst):
    e = edge_src.shape[0]
    d = table_s.shape[1]
    nw = 32
    c = 128
    n_chunks = e // c
    npw = (n_chunks + nw - 1) // nw
    mesh = plsc.VectorSubcoreMesh(core_axis_name="c", subcore_axis_name="s")

    @functools.partial(
        pl.kernel, mesh=mesh,
        out_type=[jax.ShapeDtypeStruct((e, d), F32),
                  jax.ShapeDtypeStruct((e, d), F32)],
        scratch_types=[pltpu.VMEM((c,), jnp.int32), pltpu.VMEM((c,), jnp.int32),
                       pltpu.VMEM((c, d), F32), pltpu.VMEM((c, d), F32),
                       pltpu.SemaphoreType.DMA, pltpu.SemaphoreType.DMA,
                       pltpu.SemaphoreType.DMA, pltpu.SemaphoreType.DMA,
                       pltpu.SemaphoreType.DMA, pltpu.SemaphoreType.DMA],
    )
    def gk(ts_hbm, td_hbm, src_hbm, dst_hbm, gs_hbm, gd_hbm,
           idx0, idx1, rows0, rows1, si0, si1, sg0, sg1, sw0, sw1):
        wid = lax.axis_index("s") * 2 + lax.axis_index("c")

        def one_pass(tab_hbm, ind_hbm, out_hbm):
            def issue_i(cidx, idxb, semi):
                pltpu.async_copy(ind_hbm.at[pl.ds(cidx * c, c)], idxb, semi)

            def wait_i(cidx, idxb, semi):
                pltpu.make_async_copy(ind_hbm.at[pl.ds(cidx * c, c)], idxb,
                                      semi).wait()

            def issue_g(idxb, rowsb, semg):
                pltpu.async_copy(tab_hbm.at[idxb], rowsb, semg)

            def wait_g(idxb, rowsb, semg):
                pltpu.make_async_copy(tab_hbm.at[idxb], rowsb, semg).wait()

            def issue_w(cidx, rowsb, semw):
                pltpu.async_copy(rowsb, out_hbm.at[pl.ds(cidx * c, c)], semw)

            def wait_w(cidx, rowsb, semw):
                pltpu.make_async_copy(rowsb, out_hbm.at[pl.ds(cidx * c, c)],
                                      semw).wait()

            issue_i(wid, idx0, si0)
            issue_i(wid + nw, idx1, si1)
            wait_i(wid, idx0, si0)
            issue_g(idx0, rows0, sg0)
            wait_i(wid + nw, idx1, si1)
            issue_g(idx1, rows1, sg1)
            wait_g(idx0, rows0, sg0)
            issue_w(wid, rows0, sw0)
            wait_g(idx1, rows1, sg1)
            issue_w(wid + nw, rows1, sw1)

            def body(t2, carry):
                c0 = wid + nw * 2 * t2
                c1 = c0 + nw
                wait_w(c0, rows0, sw0)
                issue_i(c0, idx0, si0)
                wait_i(c0, idx0, si0)
                issue_g(idx0, rows0, sg0)

                @pl.when(c1 < n_chunks)
                def _():
                    wait_w(c1, rows1, sw1)
                    issue_i(c1, idx1, si1)
                    wait_i(c1, idx1, si1)
                    issue_g(idx1, rows1, sg1)

                wait_g(idx0, rows0, sg0)
                issue_w(c0, rows0, sw0)

                @pl.when(c1 < n_chunks)
                def _():
                    wait_g(idx1, rows1, sg1)
                    issue_w(c1, rows1, sw1)

                return carry

            lax.fori_loop(1, npw // 2, body, 0)
            wait_w(wid, rows0, sw0)
            wait_w(wid, rows1, sw1)

        one_pass(ts_hbm, src_hbm, gs_hbm)
        one_pass(td_hbm, dst_hbm, gd_hbm)

    return gk(table_s, table_d, edge_src, edge_dst)


def _sc_scatter(emsg_t, edge_dst, n_pad):
    d, e = emsg_t.shape
    fw = 8
    c = 640
    eh = e // 2
    nc = eh // c
    mesh = plsc.VectorSubcoreMesh(core_axis_name="c", subcore_axis_name="s")

    @functools.partial(
        pl.kernel, mesh=mesh,
        out_type=jax.ShapeDtypeStruct((2, d, n_pad), F32),
        scratch_types=[pltpu.VMEM((c,), jnp.int32), pltpu.VMEM((c,), jnp.int32),
                       pltpu.VMEM((fw, c), F32), pltpu.VMEM((fw, c), F32),
                       pltpu.VMEM((fw, n_pad), F32),
                       pltpu.SemaphoreType.DMA, pltpu.SemaphoreType.DMA,
                       pltpu.SemaphoreType.DMA, pltpu.SemaphoreType.DMA],
        compiler_params=pltpu.CompilerParams(needs_layout_passes=False),
    )
    def sk(emsg_hbm, dst_hbm, out_hbm, idx0, idx1, rows0, rows1,
           acc, si0, si1, sr0, sr1):
        cid = lax.axis_index("c")
        sid = lax.axis_index("s")
        ebase = cid * eh

        def one_pass(colp):
            def zero(i, carry):
                def zcol(j, carry2):
                    acc[i, pl.ds(j * 16, 16)] = jnp.zeros((16,), F32)
                    return carry2
                return lax.fori_loop(0, n_pad // 16, zcol, carry)

            lax.fori_loop(0, fw, zero, 0)

            def issue(j, idxbuf, rowbuf, semi, semr):
                off = ebase + j * c
                pltpu.async_copy(dst_hbm.at[pl.ds(off, c)], idxbuf, semi)
                pltpu.async_copy(emsg_hbm.at[pl.ds(colp, fw), pl.ds(off, c)],
                                 rowbuf, semr)

            def drain(j, idxbuf, rowbuf, semi, semr):
                off = ebase + j * c
                pltpu.make_async_copy(dst_hbm.at[pl.ds(off, c)], idxbuf,
                                      semi).wait()
                pltpu.make_async_copy(emsg_hbm.at[pl.ds(colp, fw),
                                                  pl.ds(off, c)],
                                      rowbuf, semr).wait()

            def process(idxbuf, rowbuf):
                for g in range(c // 16):
                    dv = idxbuf[pl.ds(g * 16, 16)]
                    for cc in range(fw):
                        col = jnp.full((16,), cc, jnp.int32)
                        val = rowbuf[cc, pl.ds(g * 16, 16)]
                        plsc.addupdate_scatter(acc, [col, dv], val)

            issue(0, idx0, rows0, si0, sr0)

            def body(j2, carry):
                e0 = 2 * j2
                issue(e0 + 1, idx1, rows1, si1, sr1)
                drain(e0, idx0, rows0, si0, sr0)
                process(idx0, rows0)
                issue(e0 + 2, idx0, rows0, si0, sr0)
                drain(e0 + 1, idx1, rows1, si1, sr1)
                process(idx1, rows1)
                return carry

            lax.fori_loop(0, (nc - 1) // 2, body, 0)
            drain(nc - 1, idx0, rows0, si0, sr0)
            process(idx0, rows0)
            pltpu.sync_copy(acc, out_hbm.at[cid, pl.ds(colp, fw), :])

        one_pass(sid * 16)
        one_pass(sid * 16 + fw)

    return sk(emsg_t, edge_dst)


def _node_out_body(nmsgt_ref, ns_ref, linw, linb, out_ref):
    nmsg = (nmsgt_ref[0] + nmsgt_ref[1]).T
    out_ref[...] = _dot(nmsg, linw[...]) + linb[...] + ns_ref[...]


def _node_out(nmsg_t2, ns_pad, p, bn=1024):
    _, dn, n_pad = nmsg_t2.shape
    grid = n_pad // bn

    def row_spec(d):
        return pl.BlockSpec((bn, d), lambda i: (i, 0))

    def w_spec(a):
        return pl.BlockSpec(a.shape, lambda i: tuple(0 for _ in a.shape))

    linb = p["lin_b"].reshape(1, -1)
    return pl.pallas_call(
        _node_out_body,
        grid=(grid,),
        in_specs=[pl.BlockSpec((2, dn, bn), lambda i: (0, 0, i)), row_spec(dn),
                  w_spec(p["lin_w"]), w_spec(linb)],
        out_specs=row_spec(dn),
        out_shape=jax.ShapeDtypeStruct((n_pad, dn), F32),
    )(nmsg_t2, ns_pad, p["lin_w"], linb)


def kernel(node_fea, edge_fea, edge_sh, edge_length_embedding, edge_vec, D,
           params, edge_src, edge_dst, batch):
    p = params
    n = node_fea.shape[0]
    e = edge_fea.shape[0]
    table_s, table_d, ns = _node_prep(node_fea, p)
    gs, gd = _sc_gather(table_s, table_d, edge_src, edge_dst)
    dm = D.reshape(e, 9)
    emsg_t, edge_out = _edge_stage(edge_fea, edge_length_embedding, dm,
                                   edge_vec, gs, gd, p)
    n_pad = 10240
    nmsg_t = _sc_scatter(emsg_t, edge_dst, n_pad)
    ns_pad = jnp.pad(ns, ((0, n_pad - n), (0, 0)))
    node_out = _node_out(nmsg_t, ns_pad, p)[:n]
    return node_out, edge_out


# speedup vs baseline: 1.0507x; 1.0507x over previous
import functools
import math

import jax
import jax.numpy as jnp
from jax import lax
from jax.experimental import pallas as pl
from jax.experimental.pallas import tpu as pltpu
from jax.experimental.pallas import tpu_sc as plsc

SC = 128
H = 8
QK = 16
HD = 32
F32 = jnp.float32


def _ln(x, g, b, eps=1e-6):
    m = jnp.mean(x, axis=-1, keepdims=True)
    m2 = jnp.mean(x * x, axis=-1, keepdims=True)
    v = m2 - m * m
    return (x - m) * jax.lax.rsqrt(v + eps) * g + b


def _silu(x):
    return x * jax.nn.sigmoid(x)


def _dot(a, b):
    return jax.lax.dot(a, b, preferred_element_type=F32)


def _node_prep_body(x_ref, lng, lnb, qw1, qb1, qg, qbn, qw2, qb2,
                    kw1, kb1, kg, kbn, kw2, kb2, ws, wd, nsw, nsb,
                    ts_ref, td_ref, ns_ref):
    x = x_ref[...]
    node = _ln(x, lng[...], lnb[...])
    scal = node[:, :SC]

    def qkproj(w1, b1, g, bn, w2, b2):
        h = _silu(_ln(_dot(scal, w1[...]) + b1[...], g[...], bn[...]))
        return _dot(h, w2[...]) + b2[...]

    qn = qkproj(qw1, qb1, qg, qbn, qw2, qb2)
    kn = qkproj(kw1, kb1, kg, kbn, kw2, kb2)
    ts_ref[:, :256] = _dot(node, ws[...])
    ts_ref[:, 256:] = kn
    td_ref[:, :256] = _dot(node, wd[...])
    td_ref[:, 256:] = qn
    ns_ref[...] = _dot(x, nsw[...]) + nsb[...]


def _node_prep(node_fea, p, bn=1000):
    n, dn = node_fea.shape
    grid = n // bn

    def row_spec(d):
        return pl.BlockSpec((bn, d), lambda i: (i, 0))

    def w_spec(a):
        return pl.BlockSpec(a.shape, lambda i: tuple(0 for _ in a.shape))

    r2 = lambda a: a.reshape(1, -1)
    weights = [r2(p["ln_node_g"]), r2(p["ln_node_b"]),
               p["q_w1"], r2(p["q_b1"]), r2(p["q_g"]), r2(p["q_bn"]),
               p["q_w2"], r2(p["q_b2"]),
               p["k_w1"], r2(p["k_b1"]), r2(p["k_g"]), r2(p["k_bn"]),
               p["k_w2"], r2(p["k_b2"]),
               p["pre_w"][:dn], p["pre_w"][dn:2 * dn],
               p["ns_w"], r2(p["ns_b"])]
    return pl.pallas_call(
        _node_prep_body,
        grid=(grid,),
        in_specs=[row_spec(dn)] + [w_spec(a) for a in weights],
        out_specs=[row_spec(384), row_spec(384), row_spec(dn)],
        out_shape=[jax.ShapeDtypeStruct((n, 384), F32),
                   jax.ShapeDtypeStruct((n, 384), F32),
                   jax.ShapeDtypeStruct((n, dn), F32)],
    )(node_fea, *weights)


def _edge_body(ef_ref, el_ref, dm_ref, ev_ref, gs_ref, gd_ref,
               lneg, lneb, aw1, ab1, ag1, abg1, aw2, ab2, ag2, abg2, aw3, ab3,
               we, preb, c1r1, c1r2, c1w, c1d, c1g, c1b,
               c2r1, c2r2, c2w, c2d, onw, onb, linew, lineb, esw, esb,
               sel, expm, emsg_ref, eout_ref):
    ef = ef_ref[...]
    el = el_ref[...]
    edge = _ln(ef, lneg[...], lneb[...])
    h = _silu(_ln(_dot(el, aw1[...]) + ab1[...], ag1[...], abg1[...]))
    h = _silu(_ln(_dot(h, aw2[...]) + ab2[...], ag2[...], abg2[...]))
    bias = _dot(h, aw3[...]) + ab3[...]
    gs = gs_ref[...]
    gd = gd_ref[...]
    qk = _dot(gs[:, 256:] * gd[:, 256:], sel[...]) * (1.0 / math.sqrt(QK))
    alpha = qk + bias
    msg = gs[:, :256] + gd[:, :256] + _dot(edge, we[...]) + preb[...]
    dm = dm_ref[...]
    r1 = _dot(_silu(_dot(el, c1r1[...])), c1r2[...])
    v = _dot(msg * r1, c1w[...]) + _dot(dm, c1d[...])
    v = _silu(_ln(v, c1g[...], c1b[...]))
    r2 = _dot(_silu(_dot(el, c2r1[...])), c2r2[...])
    value = _dot(v * r2, c2w[...]) + _dot(dm, c2d[...])
    av = _dot(alpha, expm[...])
    ev = ev_ref[...]
    ons = (ev[:, 0:1] * ev[:, 0:1] + ev[:, 1:2] * ev[:, 1:2]
           + ev[:, 2:3] * ev[:, 2:3]) < 1e-20
    emsg = value * av
    emsg_ref[...] = emsg.T
    eout_ref[...] = (_dot(emsg, linew[...]) + lineb[...]
                     + _dot(ef, esw[...]) + esb[...])

    @pl.when(jnp.any(ons))
    def _():
        onsite_val = _dot(msg, onw[...]) + onb[...]
        emsg2 = jnp.where(ons, onsite_val, value) * av
        emsg_ref[...] = emsg2.T
        eout_ref[...] = (_dot(emsg2, linew[...]) + lineb[...]
                         + _dot(ef, esw[...]) + esb[...])


def _edge_stage(edge_fea, elen, dm, edge_vec, gs, gd, p, be=1280):
    e, de = edge_fea.shape
    grid = e // be
    sel = (jnp.arange(SC)[:, None] // QK == jnp.arange(H)[None, :]).astype(F32)
    expm = (jnp.arange(H)[:, None] == jnp.arange(H * HD)[None, :] // HD).astype(F32)

    def row_spec(d):
        return pl.BlockSpec((be, d), lambda i: (i, 0))

    def w_spec(a):
        return pl.BlockSpec(a.shape, lambda i: tuple(0 for _ in a.shape))

    r2 = lambda a: a.reshape(1, -1)
    weights = [r2(p["ln_edge_g"]), r2(p["ln_edge_b"]),
               p["a_w1"], r2(p["a_b1"]), r2(p["a_g1"]), r2(p["a_bg1"]),
               p["a_w2"], r2(p["a_b2"]), r2(p["a_g2"]), r2(p["a_bg2"]),
               p["a_w3"], r2(p["a_b3"]),
               p["pre_w"][512:], r2(p["pre_b"]),
               p["c1_r1"], p["c1_r2"], p["c1_w"], p["c1_d"],
               r2(p["c1_g"]), r2(p["c1_b"]),
               p["c2_r1"], p["c2_r2"], p["c2_w"], p["c2_d"],
               p["on_w"], r2(p["on_b"]),
               p["line_w"], r2(p["line_b"]), p["es_w"], r2(p["es_b"]),
               sel, expm]
    return pl.pallas_call(
        _edge_body,
        grid=(grid,),
        in_specs=[row_spec(de), row_spec(64), row_spec(9), row_spec(3),
                  row_spec(384), row_spec(384)] + [w_spec(a) for a in weights],
        out_specs=[pl.BlockSpec((256, be), lambda i: (0, i)), row_spec(de)],
        out_shape=[jax.ShapeDtypeStruct((256, e), F32),
                   jax.ShapeDtypeStruct((e, de), F32)],
    )(edge_fea, elen, dm, edge_vec, gs, gd, *weights)


def _sc_gather(table_s, table_d, edge_src, edge_dst):
    e = edge_src.shape[0]
    d = table_s.shape[1]
    nw = 32
    c = 128
    n_chunks = e // c
    npw = (n_chunks + nw - 1) // nw
    mesh = plsc.VectorSubcoreMesh(core_axis_name="c", subcore_axis_name="s")

    @functools.partial(
        pl.kernel, mesh=mesh,
        out_type=[jax.ShapeDtypeStruct((e, d), F32),
                  jax.ShapeDtypeStruct((e, d), F32)],
        scratch_types=[pltpu.VMEM((c,), jnp.int32), pltpu.VMEM((c,), jnp.int32),
                       pltpu.VMEM((c, d), F32), pltpu.VMEM((c, d), F32),
                       pltpu.SemaphoreType.DMA, pltpu.SemaphoreType.DMA,
                       pltpu.SemaphoreType.DMA, pltpu.SemaphoreType.DMA,
                       pltpu.SemaphoreType.DMA, pltpu.SemaphoreType.DMA],
    )
    def gk(ts_hbm, td_hbm, src_hbm, dst_hbm, gs_hbm, gd_hbm,
           idx0, idx1, rows0, rows1, si0, si1, sg0, sg1, sw0, sw1):
        wid = lax.axis_index("s") * 2 + lax.axis_index("c")

        def one_pass(tab_hbm, ind_hbm, out_hbm):
            def issue_i(cidx, idxb, semi):
                pltpu.async_copy(ind_hbm.at[pl.ds(cidx * c, c)], idxb, semi)

            def wait_i(cidx, idxb, semi):
                pltpu.make_async_copy(ind_hbm.at[pl.ds(cidx * c, c)], idxb,
                                      semi).wait()

            def issue_g(idxb, rowsb, semg):
                pltpu.async_copy(tab_hbm.at[idxb], rowsb, semg)

            def wait_g(idxb, rowsb, semg):
                pltpu.make_async_copy(tab_hbm.at[idxb], rowsb, semg).wait()

            def issue_w(cidx, rowsb, semw):
                pltpu.async_copy(rowsb, out_hbm.at[pl.ds(cidx * c, c)], semw)

            def wait_w(cidx, rowsb, semw):
                pltpu.make_async_copy(rowsb, out_hbm.at[pl.ds(cidx * c, c)],
                                      semw).wait()

            issue_i(wid, idx0, si0)
            issue_i(wid + nw, idx1, si1)
            wait_i(wid, idx0, si0)
            issue_g(idx0, rows0, sg0)
            wait_i(wid + nw, idx1, si1)
            issue_g(idx1, rows1, sg1)
            wait_g(idx0, rows0, sg0)
            issue_w(wid, rows0, sw0)
            wait_g(idx1, rows1, sg1)
            issue_w(wid + nw, rows1, sw1)

            def body(t2, carry):
                c0 = wid + nw * 2 * t2
                c1 = c0 + nw
                wait_w(c0, rows0, sw0)
                issue_i(c0, idx0, si0)
                wait_i(c0, idx0, si0)
                issue_g(idx0, rows0, sg0)

                @pl.when(c1 < n_chunks)
                def _():
                    wait_w(c1, rows1, sw1)
                    issue_i(c1, idx1, si1)
                    wait_i(c1, idx1, si1)
                    issue_g(idx1, rows1, sg1)

                wait_g(idx0, rows0, sg0)
                issue_w(c0, rows0, sw0)

                @pl.when(c1 < n_chunks)
                def _():
                    wait_g(idx1, rows1, sg1)
                    issue_w(c1, rows1, sw1)

                return carry

            lax.fori_loop(1, npw // 2, body, 0)
            wait_w(wid, rows0, sw0)
            wait_w(wid, rows1, sw1)

        one_pass(ts_hbm, src_hbm, gs_hbm)
        one_pass(td_hbm, dst_hbm, gd_hbm)

    return gk(table_s, table_d, edge_src, edge_dst)


def _sc_scatter(emsg_t, edge_dst, n_pad):
    d, e = emsg_t.shape
    fw = 8
    c = 640
    eh = e // 2
    nc = eh // c
    mesh = plsc.VectorSubcoreMesh(core_axis_name="c", subcore_axis_name="s")

    @functools.partial(
        pl.kernel, mesh=mesh,
        out_type=jax.ShapeDtypeStruct((2, d, n_pad), F32),
        scratch_types=[pltpu.VMEM((c,), jnp.int32), pltpu.VMEM((c,), jnp.int32),
                       pltpu.VMEM((fw, c), F32), pltpu.VMEM((fw, c), F32),
                       pltpu.VMEM((fw, n_pad), F32),
                       pltpu.SemaphoreType.DMA, pltpu.SemaphoreType.DMA,
                       pltpu.SemaphoreType.DMA, pltpu.SemaphoreType.DMA],
        compiler_params=pltpu.CompilerParams(needs_layout_passes=False),
    )
    def sk(emsg_hbm, dst_hbm, out_hbm, idx0, idx1, rows0, rows1,
           acc, si0, si1, sr0, sr1):
        cid = lax.axis_index("c")
        sid = lax.axis_index("s")
        ebase = cid * eh

        def one_pass(colp):
            def zero(i, carry):
                def zcol(j, carry2):
                    acc[i, pl.ds(j * 16, 16)] = jnp.zeros((16,), F32)
                    return carry2
                return lax.fori_loop(0, n_pad // 16, zcol, carry)

            lax.fori_loop(0, fw, zero, 0)

            def issue(j, idxbuf, rowbuf, semi, semr):
                off = ebase + j * c
                pltpu.async_copy(dst_hbm.at[pl.ds(off, c)], idxbuf, semi)
                pltpu.async_copy(emsg_hbm.at[pl.ds(colp, fw), pl.ds(off, c)],
                                 rowbuf, semr)

            def drain(j, idxbuf, rowbuf, semi, semr):
                off = ebase + j * c
                pltpu.make_async_copy(dst_hbm.at[pl.ds(off, c)], idxbuf,
                                      semi).wait()
                pltpu.make_async_copy(emsg_hbm.at[pl.ds(colp, fw),
                                                  pl.ds(off, c)],
                                      rowbuf, semr).wait()

            def process(idxbuf, rowbuf):
                for g in range(c // 16):
                    dv = idxbuf[pl.ds(g * 16, 16)]
                    for cc in range(fw):
                        col = jnp.full((16,), cc, jnp.int32)
                        val = rowbuf[cc, pl.ds(g * 16, 16)]
                        plsc.addupdate_scatter(acc, [col, dv], val)

            issue(0, idx0, rows0, si0, sr0)

            def body(j2, carry):
                e0 = 2 * j2
                issue(e0 + 1, idx1, rows1, si1, sr1)
                drain(e0, idx0, rows0, si0, sr0)
                process(idx0, rows0)
                issue(e0 + 2, idx0, rows0, si0, sr0)
                drain(e0 + 1, idx1, rows1, si1, sr1)
                process(idx1, rows1)
                return carry

            lax.fori_loop(0, (nc - 1) // 2, body, 0)
            drain(nc - 1, idx0, rows0, si0, sr0)
            process(idx0, rows0)
            pltpu.sync_copy(acc, out_hbm.at[cid, pl.ds(colp, fw), :])

        one_pass(sid * 16)
        one_pass(sid * 16 + fw)

    return sk(emsg_t, edge_dst)


def _node_out_body(nmsgt_ref, ns_ref, linw, linb, out_ref):
    nmsg = (nmsgt_ref[0] + nmsgt_ref[1]).T
    out_ref[...] = _dot(nmsg, linw[...]) + linb[...] + ns_ref[...]


def _node_out(nmsg_t2, ns_pad, p, bn=1024):
    _, dn, n_pad = nmsg_t2.shape
    grid = n_pad // bn

    def row_spec(d):
        return pl.BlockSpec((bn, d), lambda i: (i, 0))

    def w_spec(a):
        return pl.BlockSpec(a.shape, lambda i: tuple(0 for _ in a.shape))

    linb = p["lin_b"].reshape(1, -1)
    return pl.pallas_call(
        _node_out_body,
        grid=(grid,),
        in_specs=[pl.BlockSpec((2, dn, bn), lambda i: (0, 0, i)), row_spec(dn),
                  w_spec(p["lin_w"]), w_spec(linb)],
        out_specs=row_spec(dn),
        out_shape=jax.ShapeDtypeStruct((n_pad, dn), F32),
    )(nmsg_t2, ns_pad, p["lin_w"], linb)


def kernel(node_fea, edge_fea, edge_sh, edge_length_embedding, edge_vec, D,
           params, edge_src, edge_dst, batch):
    p = params
    n = node_fea.shape[0]
    e = edge_fea.shape[0]
    table_s, table_d, ns = _node_prep(node_fea, p)
    gs, gd = _sc_gather(table_s, table_d, edge_src, edge_dst)
    dm = D.reshape(e, 9)
    emsg_t, edge_out = _edge_stage(edge_fea, edge_length_embedding, dm,
                                   edge_vec, gs, gd, p)
    n_pad = 10240
    nmsg_t = _sc_scatter(emsg_t, edge_dst, n_pad)
    ns_pad = jnp.pad(ns, ((0, n_pad - n), (0, 0)))
    node_out = _node_out(nmsg_t, ns_pad, p)[:n]
    return node_out, edge_out


# speedup vs baseline: 1.1374x; 1.0825x over previous
import functools
import math

import jax
import jax.numpy as jnp
from jax import lax
from jax.experimental import pallas as pl
from jax.experimental.pallas import tpu as pltpu
from jax.experimental.pallas import tpu_sc as plsc

SC = 128
H = 8
QK = 16
HD = 32
F32 = jnp.float32


def _ln(x, g, b, eps=1e-6):
    m = jnp.mean(x, axis=-1, keepdims=True)
    m2 = jnp.mean(x * x, axis=-1, keepdims=True)
    v = m2 - m * m
    return (x - m) * jax.lax.rsqrt(v + eps) * g + b


def _silu(x):
    return x * jax.nn.sigmoid(x)


def _dot(a, b):
    return jax.lax.dot(a, b, preferred_element_type=F32)


def _node_prep_body(x_ref, lng, lnb, qw1, qb1, qg, qbn, qw2, qb2,
                    kw1, kb1, kg, kbn, kw2, kb2, ws, wd, nsw, nsb,
                    ts_ref, td_ref, ns_ref):
    x = x_ref[...]
    node = _ln(x, lng[...], lnb[...])
    scal = node[:, :SC]

    def qkproj(w1, b1, g, bn, w2, b2):
        h = _silu(_ln(_dot(scal, w1[...]) + b1[...], g[...], bn[...]))
        return _dot(h, w2[...]) + b2[...]

    qn = qkproj(qw1, qb1, qg, qbn, qw2, qb2)
    kn = qkproj(kw1, kb1, kg, kbn, kw2, kb2)

    def pack2(full):
        rb = full.astype(jnp.bfloat16).astype(F32)
        bits = jax.lax.bitcast_convert_type(rb, jnp.int32)
        lo = jax.lax.shift_right_logical(bits[:, :128], 16)
        hi = bits[:, 128:] & jnp.int32(-65536)
        return hi | lo

    ts_ref[:, :128] = pack2(_dot(node, ws[...]))
    ts_ref[:, 128:] = jax.lax.bitcast_convert_type(kn, jnp.int32)
    td_ref[:, :128] = pack2(_dot(node, wd[...]))
    td_ref[:, 128:] = jax.lax.bitcast_convert_type(qn, jnp.int32)
    ns_ref[...] = _dot(x, nsw[...]) + nsb[...]


def _node_prep(node_fea, p, bn=1000):
    n, dn = node_fea.shape
    grid = n // bn

    def row_spec(d):
        return pl.BlockSpec((bn, d), lambda i: (i, 0))

    def w_spec(a):
        return pl.BlockSpec(a.shape, lambda i: tuple(0 for _ in a.shape))

    r2 = lambda a: a.reshape(1, -1)
    weights = [r2(p["ln_node_g"]), r2(p["ln_node_b"]),
               p["q_w1"], r2(p["q_b1"]), r2(p["q_g"]), r2(p["q_bn"]),
               p["q_w2"], r2(p["q_b2"]),
               p["k_w1"], r2(p["k_b1"]), r2(p["k_g"]), r2(p["k_bn"]),
               p["k_w2"], r2(p["k_b2"]),
               p["pre_w"][:dn], p["pre_w"][dn:2 * dn],
               p["ns_w"], r2(p["ns_b"])]
    return pl.pallas_call(
        _node_prep_body,
        grid=(grid,),
        in_specs=[row_spec(dn)] + [w_spec(a) for a in weights],
        out_specs=[row_spec(256), row_spec(256), row_spec(dn)],
        out_shape=[jax.ShapeDtypeStruct((n, 256), jnp.int32),
                   jax.ShapeDtypeStruct((n, 256), jnp.int32),
                   jax.ShapeDtypeStruct((n, dn), F32)],
    )(node_fea, *weights)


def _edge_body(ef_ref, el_ref, dm_ref, ev_ref, gs_ref, gd_ref,
               lneg, lneb, aw1, ab1, ag1, abg1, aw2, ab2, ag2, abg2, aw3, ab3,
               we, preb, c1r1, c1r2, c1w, c1d, c1g, c1b,
               c2r1, c2r2, c2w, c2d, onw, onb, linew, lineb, esw, esb,
               sel, expm, emsg_ref, eout_ref):
    ef = ef_ref[...]
    el = el_ref[...]
    edge = _ln(ef, lneg[...], lneb[...])
    h = _silu(_ln(_dot(el, aw1[...]) + ab1[...], ag1[...], abg1[...]))
    h = _silu(_ln(_dot(h, aw2[...]) + ab2[...], ag2[...], abg2[...]))
    bias = _dot(h, aw3[...]) + ab3[...]
    def unpack(ref):
        bits = ref[...]
        pk = bits[:, :128]
        lo = jax.lax.bitcast_convert_type(jax.lax.shift_left(pk, 16), F32)
        hi = jax.lax.bitcast_convert_type(pk & jnp.int32(-65536), F32)
        npre = jnp.concatenate([lo, hi], axis=1)
        qkv = jax.lax.bitcast_convert_type(bits[:, 128:], F32)
        return npre, qkv

    gs_n, ks = unpack(gs_ref)
    gd_n, qs = unpack(gd_ref)
    qk = _dot(ks * qs, sel[...]) * (1.0 / math.sqrt(QK))
    alpha = qk + bias
    msg = gs_n + gd_n + _dot(edge, we[...]) + preb[...]
    dm = dm_ref[...]
    r1 = _dot(_silu(_dot(el, c1r1[...])), c1r2[...])
    v = _dot(msg * r1, c1w[...]) + _dot(dm, c1d[...])
    v = _silu(_ln(v, c1g[...], c1b[...]))
    r2 = _dot(_silu(_dot(el, c2r1[...])), c2r2[...])
    value = _dot(v * r2, c2w[...]) + _dot(dm, c2d[...])
    av = _dot(alpha, expm[...])
    ev = ev_ref[...]
    ons = (ev[:, 0:1] * ev[:, 0:1] + ev[:, 1:2] * ev[:, 1:2]
           + ev[:, 2:3] * ev[:, 2:3]) < 1e-20
    emsg = value * av
    emsg_ref[...] = emsg.T
    eout_ref[...] = (_dot(emsg, linew[...]) + lineb[...]
                     + _dot(ef, esw[...]) + esb[...])

    @pl.when(jnp.any(ons))
    def _():
        onsite_val = _dot(msg, onw[...]) + onb[...]
        emsg2 = jnp.where(ons, onsite_val, value) * av
        emsg_ref[...] = emsg2.T
        eout_ref[...] = (_dot(emsg2, linew[...]) + lineb[...]
                         + _dot(ef, esw[...]) + esb[...])


def _edge_stage(edge_fea, elen, dm, edge_vec, gs, gd, p, be=1280):
    e, de = edge_fea.shape
    grid = e // be
    sel = (jnp.arange(SC)[:, None] // QK == jnp.arange(H)[None, :]).astype(F32)
    expm = (jnp.arange(H)[:, None] == jnp.arange(H * HD)[None, :] // HD).astype(F32)

    def row_spec(d):
        return pl.BlockSpec((be, d), lambda i: (i, 0))

    def w_spec(a):
        return pl.BlockSpec(a.shape, lambda i: tuple(0 for _ in a.shape))

    r2 = lambda a: a.reshape(1, -1)
    weights = [r2(p["ln_edge_g"]), r2(p["ln_edge_b"]),
               p["a_w1"], r2(p["a_b1"]), r2(p["a_g1"]), r2(p["a_bg1"]),
               p["a_w2"], r2(p["a_b2"]), r2(p["a_g2"]), r2(p["a_bg2"]),
               p["a_w3"], r2(p["a_b3"]),
               p["pre_w"][512:], r2(p["pre_b"]),
               p["c1_r1"], p["c1_r2"], p["c1_w"], p["c1_d"],
               r2(p["c1_g"]), r2(p["c1_b"]),
               p["c2_r1"], p["c2_r2"], p["c2_w"], p["c2_d"],
               p["on_w"], r2(p["on_b"]),
               p["line_w"], r2(p["line_b"]), p["es_w"], r2(p["es_b"]),
               sel, expm]
    return pl.pallas_call(
        _edge_body,
        grid=(grid,),
        in_specs=[row_spec(de), row_spec(64), row_spec(9), row_spec(3),
                  row_spec(256), row_spec(256)] + [w_spec(a) for a in weights],
        out_specs=[pl.BlockSpec((256, be), lambda i: (0, i)), row_spec(de)],
        out_shape=[jax.ShapeDtypeStruct((256, e), F32),
                   jax.ShapeDtypeStruct((e, de), F32)],
    )(edge_fea, elen, dm, edge_vec, gs, gd, *weights)


def _sc_gather(table_s, table_d, edge_src, edge_dst):
    e = edge_src.shape[0]
    d = table_s.shape[1]
    dt = table_s.dtype
    nw = 32
    c = 128
    n_chunks = e // c
    npw = (n_chunks + nw - 1) // nw
    mesh = plsc.VectorSubcoreMesh(core_axis_name="c", subcore_axis_name="s")

    @functools.partial(
        pl.kernel, mesh=mesh,
        out_type=[jax.ShapeDtypeStruct((e, d), dt),
                  jax.ShapeDtypeStruct((e, d), dt)],
        scratch_types=[pltpu.VMEM((c,), jnp.int32), pltpu.VMEM((c,), jnp.int32),
                       pltpu.VMEM((c, d), dt), pltpu.VMEM((c, d), dt),
                       pltpu.SemaphoreType.DMA, pltpu.SemaphoreType.DMA,
                       pltpu.SemaphoreType.DMA, pltpu.SemaphoreType.DMA,
                       pltpu.SemaphoreType.DMA, pltpu.SemaphoreType.DMA],
    )
    def gk(ts_hbm, td_hbm, src_hbm, dst_hbm, gs_hbm, gd_hbm,
           idx0, idx1, rows0, rows1, si0, si1, sg0, sg1, sw0, sw1):
        wid = lax.axis_index("s") * 2 + lax.axis_index("c")

        def one_pass(tab_hbm, ind_hbm, out_hbm):
            def issue_i(cidx, idxb, semi):
                pltpu.async_copy(ind_hbm.at[pl.ds(cidx * c, c)], idxb, semi)

            def wait_i(cidx, idxb, semi):
                pltpu.make_async_copy(ind_hbm.at[pl.ds(cidx * c, c)], idxb,
                                      semi).wait()

            def issue_g(idxb, rowsb, semg):
                pltpu.async_copy(tab_hbm.at[idxb], rowsb, semg)

            def wait_g(idxb, rowsb, semg):
                pltpu.make_async_copy(tab_hbm.at[idxb], rowsb, semg).wait()

            def issue_w(cidx, rowsb, semw):
                pltpu.async_copy(rowsb, out_hbm.at[pl.ds(cidx * c, c)], semw)

            def wait_w(cidx, rowsb, semw):
                pltpu.make_async_copy(rowsb, out_hbm.at[pl.ds(cidx * c, c)],
                                      semw).wait()

            issue_i(wid, idx0, si0)
            issue_i(wid + nw, idx1, si1)
            wait_i(wid, idx0, si0)
            issue_g(idx0, rows0, sg0)
            wait_i(wid + nw, idx1, si1)
            issue_g(idx1, rows1, sg1)
            wait_g(idx0, rows0, sg0)
            issue_w(wid, rows0, sw0)
            wait_g(idx1, rows1, sg1)
            issue_w(wid + nw, rows1, sw1)

            def body(t2, carry):
                c0 = wid + nw * 2 * t2
                c1 = c0 + nw
                wait_w(c0, rows0, sw0)
                issue_i(c0, idx0, si0)
                wait_i(c0, idx0, si0)
                issue_g(idx0, rows0, sg0)

                @pl.when(c1 < n_chunks)
                def _():
                    wait_w(c1, rows1, sw1)
                    issue_i(c1, idx1, si1)
                    wait_i(c1, idx1, si1)
                    issue_g(idx1, rows1, sg1)

                wait_g(idx0, rows0, sg0)
                issue_w(c0, rows0, sw0)

                @pl.when(c1 < n_chunks)
                def _():
                    wait_g(idx1, rows1, sg1)
                    issue_w(c1, rows1, sw1)

                return carry

            lax.fori_loop(1, npw // 2, body, 0)
            wait_w(wid, rows0, sw0)
            wait_w(wid, rows1, sw1)

        one_pass(ts_hbm, src_hbm, gs_hbm)
        one_pass(td_hbm, dst_hbm, gd_hbm)

    return gk(table_s, table_d, edge_src, edge_dst)


def _sc_scatter(emsg_t, edge_dst, n_pad):
    d, e = emsg_t.shape
    fw = 8
    c = 640
    eh = e // 2
    nc = eh // c
    mesh = plsc.VectorSubcoreMesh(core_axis_name="c", subcore_axis_name="s")

    @functools.partial(
        pl.kernel, mesh=mesh,
        out_type=jax.ShapeDtypeStruct((2, d, n_pad), F32),
        scratch_types=[pltpu.VMEM((c,), jnp.int32), pltpu.VMEM((c,), jnp.int32),
                       pltpu.VMEM((fw, c), F32), pltpu.VMEM((fw, c), F32),
                       pltpu.VMEM((fw, n_pad), F32),
                       pltpu.SemaphoreType.DMA, pltpu.SemaphoreType.DMA,
                       pltpu.SemaphoreType.DMA, pltpu.SemaphoreType.DMA],
        compiler_params=pltpu.CompilerParams(needs_layout_passes=False),
    )
    def sk(emsg_hbm, dst_hbm, out_hbm, idx0, idx1, rows0, rows1,
           acc, si0, si1, sr0, sr1):
        cid = lax.axis_index("c")
        sid = lax.axis_index("s")
        ebase = cid * eh

        def one_pass(colp):
            def zero(i, carry):
                def zcol(j, carry2):
                    acc[i, pl.ds(j * 16, 16)] = jnp.zeros((16,), F32)
                    return carry2
                return lax.fori_loop(0, n_pad // 16, zcol, carry)

            lax.fori_loop(0, fw, zero, 0)

            def issue(j, idxbuf, rowbuf, semi, semr):
                off = ebase + j * c
                pltpu.async_copy(dst_hbm.at[pl.ds(off, c)], idxbuf, semi)
                pltpu.async_copy(emsg_hbm.at[pl.ds(colp, fw), pl.ds(off, c)],
                                 rowbuf, semr)

            def drain(j, idxbuf, rowbuf, semi, semr):
                off = ebase + j * c
                pltpu.make_async_copy(dst_hbm.at[pl.ds(off, c)], idxbuf,
                                      semi).wait()
                pltpu.make_async_copy(emsg_hbm.at[pl.ds(colp, fw),
                                                  pl.ds(off, c)],
                                      rowbuf, semr).wait()

            def process(idxbuf, rowbuf):
                for g in range(c // 16):
                    dv = idxbuf[pl.ds(g * 16, 16)]
                    for cc in range(fw):
                        col = jnp.full((16,), cc, jnp.int32)
                        val = rowbuf[cc, pl.ds(g * 16, 16)]
                        plsc.addupdate_scatter(acc, [col, dv], val)

            issue(0, idx0, rows0, si0, sr0)

            def body(j2, carry):
                e0 = 2 * j2
                issue(e0 + 1, idx1, rows1, si1, sr1)
                drain(e0, idx0, rows0, si0, sr0)
                process(idx0, rows0)
                issue(e0 + 2, idx0, rows0, si0, sr0)
                drain(e0 + 1, idx1, rows1, si1, sr1)
                process(idx1, rows1)
                return carry

            lax.fori_loop(0, (nc - 1) // 2, body, 0)
            drain(nc - 1, idx0, rows0, si0, sr0)
            process(idx0, rows0)
            pltpu.sync_copy(acc, out_hbm.at[cid, pl.ds(colp, fw), :])

        one_pass(sid * 16)
        one_pass(sid * 16 + fw)

    return sk(emsg_t, edge_dst)


def _node_out_body(nmsgt_ref, ns_ref, linw, linb, out_ref):
    nmsg = (nmsgt_ref[0] + nmsgt_ref[1]).T
    out_ref[...] = _dot(nmsg, linw[...]) + linb[...] + ns_ref[...]


def _node_out(nmsg_t2, ns_pad, p, bn=1024):
    _, dn, n_pad = nmsg_t2.shape
    grid = n_pad // bn

    def row_spec(d):
        return pl.BlockSpec((bn, d), lambda i: (i, 0))

    def w_spec(a):
        return pl.BlockSpec(a.shape, lambda i: tuple(0 for _ in a.shape))

    linb = p["lin_b"].reshape(1, -1)
    return pl.pallas_call(
        _node_out_body,
        grid=(grid,),
        in_specs=[pl.BlockSpec((2, dn, bn), lambda i: (0, 0, i)), row_spec(dn),
                  w_spec(p["lin_w"]), w_spec(linb)],
        out_specs=row_spec(dn),
        out_shape=jax.ShapeDtypeStruct((n_pad, dn), F32),
    )(nmsg_t2, ns_pad, p["lin_w"], linb)


def kernel(node_fea, edge_fea, edge_sh, edge_length_embedding, edge_vec, D,
           params, edge_src, edge_dst, batch):
    p = params
    n = node_fea.shape[0]
    e = edge_fea.shape[0]
    table_s, table_d, ns = _node_prep(node_fea, p)
    gs, gd = _sc_gather(table_s, table_d, edge_src, edge_dst)
    dm = D.reshape(e, 9)
    emsg_t, edge_out = _edge_stage(edge_fea, edge_length_embedding, dm,
                                   edge_vec, gs, gd, p)
    n_pad = 10240
    nmsg_t = _sc_scatter(emsg_t, edge_dst, n_pad)
    ns_pad = jnp.pad(ns, ((0, n_pad - n), (0, 0)))
    node_out = _node_out(nmsg_t, ns_pad, p)[:n]
    return node_out, edge_out
